# baseline (speedup 1.0000x reference)
import functools

import jax
import jax.numpy as jnp
from jax import lax
from jax.experimental import pallas as pl
from jax.experimental.pallas import tpu as pltpu
from jax.experimental.pallas import tpu_sc as plsc

_N = 10000
_E = 320000
_H = 128
_ED = 3

_NC = 2
_NS = 16
_NW = _NC * _NS
_EPW = _E // _NW
_CH = 80
_NCHUNK = _EPW // _CH
_SBLK = 25
_NSTAGE = _NCHUNK // _SBLK
_SLC = 624
_TAIL = _N - _NS * _SLC
_RB = 48

_MESH = plsc.VectorSubcoreMesh(
    core_axis_name="c", subcore_axis_name="s", num_cores=_NC, num_subcores=_NS
)


def _spmm_body(mode, *refs):
    dega_h = degb_h = None
    if mode == "attn":
        (ei_h, ae_h, ad_h, as_h, tbl_h, zr_h, outa_h, outb_h,
         ad_v, as_v, sidx, didx, ae_v, rows0, rows1, acc, gsem0, gsem1) = refs
    elif mode == "plain0":
        (ei_h, tbl_h, zr_h, outa_h, outb_h,
         sidx, didx, rows0, rows1, acc, gsem0, gsem1) = refs
    else:
        (ei_h, tbl_h, zr_h, outa_h, outb_h, dega_h, degb_h,
         sidx, didx, rows0, rows1, hist_v, rbuf, degv, acc, hists_sh,
         gsem0, gsem1) = refs
    rows = (rows0, rows1)
    gsem = (gsem0, gsem1)

    c = lax.axis_index("c")
    s = lax.axis_index("s")
    wid = c * _NS + s

    if mode == "attn":
        pltpu.sync_copy(ad_h, ad_v)
        pltpu.sync_copy(as_h, as_v)

    row0 = pl.multiple_of(s * _SLC, 8)
    pltpu.sync_copy(zr_h, acc.at[pl.ds(row0, _SLC)])

    @pl.when(s == 0)
    def _():
        pltpu.sync_copy(zr_h.at[pl.ds(0, _TAIL)], acc.at[pl.ds(_NS * _SLC, _TAIL)])

    if mode == "plain":
        def zhist(r, carry):
            hist_v[pl.ds(pl.multiple_of(r * 16, 16), 16)] = jnp.zeros(
                (16,), jnp.float32)
            return carry
        lax.fori_loop(0, _N // 16, zhist, 0)

    plsc.subcore_barrier()

    def start_gather(ci, b):
        pltpu.async_copy(tbl_h.at[sidx.at[ci]], rows[b], gsem[b])

    def wait_gather(ci, b):
        pltpu.make_async_copy(tbl_h.at[sidx.at[ci]], rows[b], gsem[b]).wait()

    ones16 = jnp.ones((16,), jnp.float32)

    def process(ci, b):
        rb = rows[b]
        if mode == "plain":
            for g in range(_CH // 16):
                di = didx[ci, pl.ds(g * 16, 16)]
                plsc.addupdate_scatter(hist_v, [di], ones16)
        elif mode == "attn":
            @plsc.parallel_loop(0, _CH // 16, step=1, unroll=_CH // 16,
                                carry=jnp.int32(0))
            def _(g, cval):
                off = pl.multiple_of(g * 16, 16)
                si = sidx[ci, pl.ds(off, 16)]
                di = didx[ci, pl.ds(off, 16)]
                av = (plsc.load_gather(ad_v, [di])
                      + plsc.load_gather(as_v, [si])
                      + ae_v[pl.ds(ci * _CH + off, 16)])
                av = jnp.maximum(av, 0.01 * av)
                av = 1.0 / (1.0 + jnp.exp(-av))
                for l in range(16):
                    e = off + l
                    wsc = av[l]
                    for j in range(_H // 16):
                        rb[e, pl.ds(j * 16, 16)] = rb[e, pl.ds(j * 16, 16)] * wsc
                return cval
        pltpu.sync_copy(rb, acc.at[didx.at[ci]], add=True)

    def block(blk, carry):
        cb = blk * _SBLK
        pltpu.sync_copy(ei_h.at[0, wid, pl.ds(cb, _SBLK)], sidx)
        pltpu.sync_copy(ei_h.at[1, wid, pl.ds(cb, _SBLK)], didx)
        if mode == "attn":
            ebase = pl.multiple_of(wid * _EPW + cb * _CH, 8)
            pltpu.sync_copy(ae_h.at[pl.ds(ebase, _SBLK * _CH)], ae_v)
        start_gather(0, 0)

        def pair(i, carry2):
            ci0 = i * 2
            start_gather(ci0 + 1, 1)
            wait_gather(ci0, 0)
            process(ci0, 0)
            start_gather(ci0 + 2, 0)
            wait_gather(ci0 + 1, 1)
            process(ci0 + 1, 1)
            return carry2

        lax.fori_loop(0, (_SBLK - 1) // 2, pair, 0)
        wait_gather(_SBLK - 1, 0)
        process(_SBLK - 1, 0)
        return carry

    lax.fori_loop(0, _NSTAGE, block, 0)

    if mode == "plain":
        pltpu.sync_copy(hist_v, hists_sh.at[s])
    plsc.subcore_barrier()

    def copy_out(out_h, deg_h):
        pltpu.sync_copy(acc.at[pl.ds(row0, _SLC)], out_h.at[pl.ds(row0, _SLC)])

        @pl.when(s == 0)
        def _():
            pltpu.sync_copy(acc.at[pl.ds(_NS * _SLC, _TAIL)],
                            out_h.at[pl.ds(_NS * _SLC, _TAIL)])

        if mode == "plain":
            for k in range(_SLC // _RB):
                col = pl.multiple_of(row0 + k * _RB, 8)
                pltpu.sync_copy(hists_sh.at[:, pl.ds(col, _RB)], rbuf)
                for j in range(_RB // 16):
                    tot = rbuf[0, pl.ds(j * 16, 16)]
                    for r in range(1, _NS):
                        tot = tot + rbuf[r, pl.ds(j * 16, 16)]
                    degv[pl.ds(k * _RB + j * 16, 16)] = tot
            pltpu.sync_copy(degv, deg_h.at[pl.ds(row0, _SLC)])

            @pl.when(s == 0)
            def _():
                pltpu.sync_copy(hists_sh.at[:, pl.ds(_NS * _SLC, _TAIL)],
                                rbuf.at[:, pl.ds(0, _TAIL)])
                tot = rbuf[0, pl.ds(0, 16)]
                for r in range(1, _NS):
                    tot = tot + rbuf[r, pl.ds(0, 16)]
                degv[pl.ds(0, 16)] = tot
                pltpu.sync_copy(degv.at[pl.ds(0, _TAIL)],
                                deg_h.at[pl.ds(_NS * _SLC, _TAIL)])

    @pl.when(c == 0)
    def _():
        copy_out(outa_h, dega_h if mode == "plain" else None)

    @pl.when(c == 1)
    def _():
        copy_out(outb_h, degb_h if mode == "plain" else None)


def _make_spmm(mode):
    scratch = []
    if mode == "attn":
        scratch += [pltpu.VMEM((_N,), jnp.float32), pltpu.VMEM((_N,), jnp.float32)]
    scratch += [pltpu.VMEM((_SBLK, _CH), jnp.int32),
                pltpu.VMEM((_SBLK, _CH), jnp.int32)]
    if mode == "attn":
        scratch += [pltpu.VMEM((_SBLK * _CH,), jnp.float32)]
    scratch += [
        pltpu.VMEM((_CH, _H), jnp.float32),
        pltpu.VMEM((_CH, _H), jnp.float32),
    ]
    if mode == "plain":
        scratch += [
            pltpu.VMEM((_N,), jnp.float32),
            pltpu.VMEM((_NS, _RB), jnp.float32),
            pltpu.VMEM((_SLC,), jnp.float32),
        ]
    scratch += [pltpu.VMEM_SHARED((_N, _H), jnp.float32)]
    if mode == "plain":
        scratch += [pltpu.VMEM_SHARED((_NS, _N), jnp.float32)]
    scratch += [pltpu.SemaphoreType.DMA, pltpu.SemaphoreType.DMA]
    part = jax.ShapeDtypeStruct((_N, _H), jnp.float32)
    out_type = [part, part]
    if mode == "plain":
        degp = jax.ShapeDtypeStruct((_N,), jnp.float32)
        out_type = [part, part, degp, degp]
    return pl.kernel(
        functools.partial(_spmm_body, mode),
        out_type=out_type,
        mesh=_MESH,
        scratch_types=scratch,
        compiler_params=pltpu.CompilerParams(
            needs_layout_passes=False, use_tc_tiling_on_sc=False),
    )


_spmm_attn = _make_spmm("attn")
_spmm_plain = _make_spmm("plain")
_spmm_plain0 = _make_spmm("plain0")


_BN = 1000
_GN = _N // _BN
_BM = 1024
_GM = -(-_N // _BM)
_BE = 16384
_GE = -(-_E // _BE)


def _row_spec(width, bn=_BN):
    return pl.BlockSpec((bn, width), lambda i: (i, 0))


def _vec_spec(bn=_BN):
    return pl.BlockSpec((bn,), lambda i: (i,))


def _const_spec(shape):
    nd = len(shape)
    return pl.BlockSpec(shape, lambda i: (0,) * nd)


def _dot(a, b):
    return jnp.dot(a, b, preferred_element_type=jnp.float32)


def _tca_body(x_ref, wne_ref, bne_ref, wl0_ref, bl0_ref, watt_ref, batt_ref,
              h_ref, hl_ref, ad_ref, as_ref):
    h = _dot(x_ref[...], wne_ref[...]) + bne_ref[...]
    hl = _dot(h, wl0_ref[...]) + bl0_ref[...]
    wa = watt_ref[...]
    h_ref[...] = h
    hl_ref[...] = hl
    ad_ref[...] = jnp.sum(hl * wa[0:_H, 0], axis=1) + batt_ref[...][0]
    as_ref[...] = jnp.sum(hl * wa[_H:2 * _H, 0], axis=1)


_tca = pl.pallas_call(
    _tca_body,
    grid=(_GM,),
    in_specs=[
        _row_spec(_H, _BM),
        _const_spec((_H, _H)), _const_spec((1, _H)),
        _const_spec((_H, _H)), _const_spec((1, _H)),
        _const_spec((2 * _H + _ED, 1)), _const_spec((1,)),
    ],
    out_specs=[_row_spec(_H, _BM), _row_spec(_H, _BM), _vec_spec(_BM),
               _vec_spec(_BM)],
    out_shape=[
        jax.ShapeDtypeStruct((_N, _H), jnp.float32),
        jax.ShapeDtypeStruct((_N, _H), jnp.float32),
        jax.ShapeDtypeStruct((_N,), jnp.float32),
        jax.ShapeDtypeStruct((_N,), jnp.float32),
    ],
)


def _tca2_body(ea_ref, watt_ref, ae_ref):
    ea = ea_ref[...]
    wa = watt_ref[...]
    ae_ref[...] = (ea[0] * wa[2 * _H, 0] + ea[1] * wa[2 * _H + 1, 0]
                   + ea[2] * wa[2 * _H + 2, 0])


_tca2 = pl.pallas_call(
    _tca2_body,
    grid=(_GE,),
    in_specs=[pl.BlockSpec((_ED, _BE), lambda i: (0, i)),
              _const_spec((2 * _H + _ED, 1))],
    out_specs=pl.BlockSpec((_BE,), lambda i: (i,)),
    out_shape=jax.ShapeDtypeStruct((_E,), jnp.float32),
)


def _stats_update(st_ref, v):
    @pl.when(pl.program_id(0) == 0)
    def _():
        st_ref[...] = jnp.zeros((8, _H), jnp.float32)

    upd = jnp.concatenate(
        [jnp.sum(v, axis=0)[None, :], jnp.sum(v * v, axis=0)[None, :],
         jnp.zeros((6, _H), jnp.float32)], axis=0)
    st_ref[...] += upd


def _bn_apply(st_ref, v, g, b):
    mu = st_ref[0:1, :] / _N
    var = st_ref[1:2, :] / _N - mu * mu
    return g * (v - mu) * lax.rsqrt(var + 1e-5) + b


def _tcb1_body(pa_ref, pb_ref, s_ref, st_ref):
    v = pa_ref[...] + pb_ref[...]
    s_ref[...] = v
    _stats_update(st_ref, v)


_tcb1 = pl.pallas_call(
    _tcb1_body,
    grid=(_GN,),
    in_specs=[_row_spec(_H), _row_spec(_H)],
    out_specs=[_row_spec(_H), _const_spec((8, _H))],
    out_shape=[jax.ShapeDtypeStruct((_N, _H), jnp.float32),
               jax.ShapeDtypeStruct((8, _H), jnp.float32)],
)


def _tcb2_body(s_ref, st_ref, h_ref, g_ref, b_ref, hb_ref):
    y = _bn_apply(st_ref, s_ref[...], g_ref[...], b_ref[...])
    hb_ref[...] = jnp.maximum(y, 0.0) + h_ref[...]


_tcb2 = pl.pallas_call(
    _tcb2_body,
    grid=(_GN,),
    in_specs=[_row_spec(_H), _const_spec((8, _H)), _row_spec(_H),
              _const_spec((1, _H)), _const_spec((1, _H))],
    out_specs=_row_spec(_H),
    out_shape=jax.ShapeDtypeStruct((_N, _H), jnp.float32),
)


def _tcc1_body(pa_ref, pb_ref, dega_ref, degb_ref, hb_ref, wsl_ref, bsl_ref,
               wsr_ref, t_ref, dinvc_ref, dinvf_ref, st_ref):
    ssum = pa_ref[...] + pb_ref[...]
    deg = dega_ref[...] + degb_ref[...]
    agg = ssum / jnp.maximum(deg, 1.0)[:, None]
    t = _dot(agg, wsl_ref[...]) + bsl_ref[...] + _dot(hb_ref[...], wsr_ref[...])
    t_ref[...] = t
    dinv = lax.rsqrt(deg + 1.0)
    dinvc_ref[...] = dinv[:, None]
    dinvf_ref[...] = dinv
    rows = pl.program_id(0) * _BM + lax.broadcasted_iota(jnp.int32, (_BM, 1), 0)
    _stats_update(st_ref, jnp.where(rows < _N, t, 0.0))


_tcc1 = pl.pallas_call(
    _tcc1_body,
    grid=(_GM,),
    in_specs=[_row_spec(_H, _BM), _row_spec(_H, _BM),
              _vec_spec(_BM), _vec_spec(_BM),
              _row_spec(_H, _BM), _const_spec((_H, _H)), _const_spec((1, _H)),
              _const_spec((_H, _H))],
    out_specs=[_row_spec(_H, _BM), _row_spec(1, _BM), _vec_spec(_BM),
               _const_spec((8, _H))],
    out_shape=[jax.ShapeDtypeStruct((_N, _H), jnp.float32),
               jax.ShapeDtypeStruct((_N, 1), jnp.float32),
               jax.ShapeDtypeStruct((_N,), jnp.float32),
               jax.ShapeDtypeStruct((8, _H), jnp.float32)],
)


def _tcc2_body(t_ref, st_ref, hb_ref, dinv_ref, g_ref, b_ref, wgcn_ref,
               hc_ref, hwp_ref):
    y = _bn_apply(st_ref, t_ref[...], g_ref[...], b_ref[...])
    hc = jnp.maximum(y, 0.0) + hb_ref[...]
    hc_ref[...] = hc
    hwp_ref[...] = _dot(hc, wgcn_ref[...]) * dinv_ref[...]


_tcc2 = pl.pallas_call(
    _tcc2_body,
    grid=(_GN,),
    in_specs=[_row_spec(_H), _const_spec((8, _H)), _row_spec(_H),
              _row_spec(1),
              _const_spec((1, _H)), _const_spec((1, _H)), _const_spec((_H, _H))],
    out_specs=[_row_spec(_H), _row_spec(_H)],
    out_shape=[jax.ShapeDtypeStruct((_N, _H), jnp.float32),
               jax.ShapeDtypeStruct((_N, _H), jnp.float32)],
)


def _tcd1_body(pa_ref, pb_ref, hwp_ref, dinv_ref, bgcn_ref, s_ref, st_ref):
    dinv = dinv_ref[...]
    v = (pa_ref[...] + pb_ref[...] + hwp_ref[...]) * dinv + bgcn_ref[...]
    s_ref[...] = v
    _stats_update(st_ref, v)


_tcd1 = pl.pallas_call(
    _tcd1_body,
    grid=(_GN,),
    in_specs=[_row_spec(_H), _row_spec(_H),
              _row_spec(_H), _row_spec(1), _const_spec((1, _H))],
    out_specs=[_row_spec(_H), _const_spec((8, _H))],
    out_shape=[jax.ShapeDtypeStruct((_N, _H), jnp.float32),
               jax.ShapeDtypeStruct((8, _H), jnp.float32)],
)


def _tcd2_body(s_ref, st_ref, hc_ref, g_ref, b_ref, wr1_ref, br1_ref,
               wr2_ref, br2_ref, wr3_ref, br3_ref, out_ref):
    y = _bn_apply(st_ref, s_ref[...], g_ref[...], b_ref[...])
    h = jnp.maximum(y, 0.0) + hc_ref[...]
    r = jnp.maximum(_dot(h, wr1_ref[...]) + br1_ref[...], 0.0)
    r = jnp.maximum(_dot(r, wr2_ref[...]) + br2_ref[...], 0.0)
    out_ref[...] = _dot(r, wr3_ref[...]) + br3_ref[...]


_tcd2 = pl.pallas_call(
    _tcd2_body,
    grid=(_GN,),
    in_specs=[_row_spec(_H), _const_spec((8, _H)), _row_spec(_H),
              _const_spec((1, _H)), _const_spec((1, _H)),
              _const_spec((_H, _H)), _const_spec((1, _H)),
              _const_spec((_H, _H // 2)), _const_spec((1, _H // 2)),
              _const_spec((_H // 2, 1)), _const_spec((1, 1))],
    out_specs=_row_spec(1),
    out_shape=jax.ShapeDtypeStruct((_N, 1), jnp.float32),
)


def kernel(x, edge_index, edge_attr, W_ne, b_ne, W_ee, b_ee, W_lin0, b_lin0,
           W_att, b_att, bn0_g, bn0_b, W_sl, b_sl, W_sr, bn1_g, bn1_b,
           W_gcn, b_gcn, bn2_g, bn2_b, W_r1, b_r1, W_r2, b_r2, W_r3, b_r3):
    del W_ee, b_ee
    ei = edge_index.reshape(2, _NW, _NCHUNK, _CH)
    row1 = lambda v: v.reshape(1, -1)
    zrows = jnp.zeros((_SLC, _H), jnp.float32)

    h, hl, a_d, a_s = _tca(x, W_ne, row1(b_ne), W_lin0, row1(b_lin0),
                           W_att, b_att)
    ae = _tca2(edge_attr.T, W_att)

    p0a, p0b = _spmm_attn(ei, ae, a_d, a_s, hl, zrows)
    s0, st0 = _tcb1(p0a, p0b)
    hb = _tcb2(s0, st0, h, row1(bn0_g), row1(bn0_b))

    p1a, p1b, dega, degb = _spmm_plain(ei, hb, zrows)
    t, dinv_c, dinv_f, st1 = _tcc1(p1a, p1b, dega, degb, hb,
                                   W_sl, row1(b_sl), W_sr)
    del dinv_f
    hc, hwp = _tcc2(t, st1, hb, dinv_c, row1(bn1_g), row1(bn1_b), W_gcn)

    p2a, p2b = _spmm_plain0(ei, hwp, zrows)
    s2, st2 = _tcd1(p2a, p2b, hwp, dinv_c, row1(b_gcn))
    out = _tcd2(s2, st2, hc, row1(bn2_g), row1(bn2_b), W_r1, row1(b_r1),
                W_r2, row1(b_r2), W_r3, row1(b_r3))
    return out


# speedup vs baseline: 1.0645x; 1.0645x over previous
import functools

import jax
import jax.numpy as jnp
from jax import lax
from jax.experimental import pallas as pl
from jax.experimental.pallas import tpu as pltpu
from jax.experimental.pallas import tpu_sc as plsc

_N = 10000
_E = 320000
_H = 128
_ED = 3

_NC = 2
_NS = 16
_NW = _NC * _NS
_EPW = _E // _NW
_CH = 80
_NCHUNK = _EPW // _CH
_SBLK = 25
_NSTAGE = _NCHUNK // _SBLK
_SLC = 624
_TAIL = _N - _NS * _SLC
_RB = 48

_MESH = plsc.VectorSubcoreMesh(
    core_axis_name="c", subcore_axis_name="s", num_cores=_NC, num_subcores=_NS
)


def _spmm_body(mode, *refs):
    dega_h = degb_h = None
    if mode == "attn":
        (ei_h, ae_h, ad_h, as_h, tbl_h, zr_h, outa_h, outb_h,
         ad_v, as_v, sidx, didx, ae_v, rows0, rows1, acc, gsem0, gsem1) = refs
    elif mode == "plain0":
        (ei_h, tbl_h, zr_h, outa_h, outb_h,
         sidx, didx, rows0, rows1, acc, gsem0, gsem1) = refs
    else:
        (ei_h, tbl_h, zr_h, outa_h, outb_h, dega_h, degb_h,
         sidx, didx, rows0, rows1, hist_v, rbuf, degv, acc, hists_sh,
         gsem0, gsem1) = refs
    rows = (rows0, rows1)
    gsem = (gsem0, gsem1)

    c = lax.axis_index("c")
    s = lax.axis_index("s")
    wid = c * _NS + s

    if mode == "attn":
        pltpu.sync_copy(ad_h, ad_v)
        pltpu.sync_copy(as_h, as_v)

    row0 = pl.multiple_of(s * _SLC, 8)
    pltpu.sync_copy(zr_h, acc.at[pl.ds(row0, _SLC)])

    @pl.when(s == 0)
    def _():
        pltpu.sync_copy(zr_h.at[pl.ds(0, _TAIL)], acc.at[pl.ds(_NS * _SLC, _TAIL)])

    if mode == "plain":
        def zhist(r, carry):
            hist_v[pl.ds(pl.multiple_of(r * 16, 16), 16)] = jnp.zeros(
                (16,), jnp.float32)
            return carry
        lax.fori_loop(0, _N // 16, zhist, 0)

    plsc.subcore_barrier()

    def start_gather(ci, b):
        pltpu.async_copy(tbl_h.at[sidx.at[ci]], rows[b], gsem[b])

    def wait_gather(ci, b):
        pltpu.make_async_copy(tbl_h.at[sidx.at[ci]], rows[b], gsem[b]).wait()

    ones16 = jnp.ones((16,), jnp.float32)

    def process(ci, b):
        rb = rows[b]
        if mode == "plain":
            for g in range(_CH // 16):
                di = didx[ci, pl.ds(g * 16, 16)]
                plsc.addupdate_scatter(hist_v, [di], ones16)
        elif mode == "attn":
            @plsc.parallel_loop(0, _CH // 16, step=1, carry=jnp.int32(0))
            def _(g, cval):
                off = pl.multiple_of(g * 16, 16)
                si = sidx[ci, pl.ds(off, 16)]
                di = didx[ci, pl.ds(off, 16)]
                av = (plsc.load_gather(ad_v, [di])
                      + plsc.load_gather(as_v, [si])
                      + ae_v[pl.ds(ci * _CH + off, 16)])
                av = jnp.maximum(av, 0.01 * av)
                av = 1.0 / (1.0 + jnp.exp(-av))
                for l in range(16):
                    e = off + l
                    wsc = av[l]
                    for j in range(_H // 16):
                        rb[e, pl.ds(j * 16, 16)] = rb[e, pl.ds(j * 16, 16)] * wsc
                return cval
        pltpu.sync_copy(rb, acc.at[didx.at[ci]], add=True)

    def block(blk, carry):
        cb = blk * _SBLK
        pltpu.sync_copy(ei_h.at[0, wid, pl.ds(cb, _SBLK)], sidx)
        pltpu.sync_copy(ei_h.at[1, wid, pl.ds(cb, _SBLK)], didx)
        if mode == "attn":
            ebase = pl.multiple_of(wid * _EPW + cb * _CH, 8)
            pltpu.sync_copy(ae_h.at[pl.ds(ebase, _SBLK * _CH)], ae_v)
        start_gather(0, 0)

        def pair(i, carry2):
            ci0 = i * 2
            start_gather(ci0 + 1, 1)
            wait_gather(ci0, 0)
            process(ci0, 0)
            start_gather(ci0 + 2, 0)
            wait_gather(ci0 + 1, 1)
            process(ci0 + 1, 1)
            return carry2

        lax.fori_loop(0, (_SBLK - 1) // 2, pair, 0)
        wait_gather(_SBLK - 1, 0)
        process(_SBLK - 1, 0)
        return carry

    lax.fori_loop(0, _NSTAGE, block, 0)

    if mode == "plain":
        pltpu.sync_copy(hist_v, hists_sh.at[s])
    plsc.subcore_barrier()

    def copy_out(out_h, deg_h):
        pltpu.sync_copy(acc.at[pl.ds(row0, _SLC)], out_h.at[pl.ds(row0, _SLC)])

        @pl.when(s == 0)
        def _():
            pltpu.sync_copy(acc.at[pl.ds(_NS * _SLC, _TAIL)],
                            out_h.at[pl.ds(_NS * _SLC, _TAIL)])

        if mode == "plain":
            for k in range(_SLC // _RB):
                col = pl.multiple_of(row0 + k * _RB, 8)
                pltpu.sync_copy(hists_sh.at[:, pl.ds(col, _RB)], rbuf)
                for j in range(_RB // 16):
                    tot = rbuf[0, pl.ds(j * 16, 16)]
                    for r in range(1, _NS):
                        tot = tot + rbuf[r, pl.ds(j * 16, 16)]
                    degv[pl.ds(k * _RB + j * 16, 16)] = tot
            pltpu.sync_copy(degv, deg_h.at[pl.ds(row0, _SLC)])

            @pl.when(s == 0)
            def _():
                pltpu.sync_copy(hists_sh.at[:, pl.ds(_NS * _SLC, _TAIL)],
                                rbuf.at[:, pl.ds(0, _TAIL)])
                tot = rbuf[0, pl.ds(0, 16)]
                for r in range(1, _NS):
                    tot = tot + rbuf[r, pl.ds(0, 16)]
                degv[pl.ds(0, 16)] = tot
                pltpu.sync_copy(degv.at[pl.ds(0, _TAIL)],
                                deg_h.at[pl.ds(_NS * _SLC, _TAIL)])

    @pl.when(c == 0)
    def _():
        copy_out(outa_h, dega_h if mode == "plain" else None)

    @pl.when(c == 1)
    def _():
        copy_out(outb_h, degb_h if mode == "plain" else None)


def _make_spmm(mode):
    scratch = []
    if mode == "attn":
        scratch += [pltpu.VMEM((_N,), jnp.float32), pltpu.VMEM((_N,), jnp.float32)]
    scratch += [pltpu.VMEM((_SBLK, _CH), jnp.int32),
                pltpu.VMEM((_SBLK, _CH), jnp.int32)]
    if mode == "attn":
        scratch += [pltpu.VMEM((_SBLK * _CH,), jnp.float32)]
    scratch += [
        pltpu.VMEM((_CH, _H), jnp.float32),
        pltpu.VMEM((_CH, _H), jnp.float32),
    ]
    if mode == "plain":
        scratch += [
            pltpu.VMEM((_N,), jnp.float32),
            pltpu.VMEM((_NS, _RB), jnp.float32),
            pltpu.VMEM((_SLC,), jnp.float32),
        ]
    scratch += [pltpu.VMEM_SHARED((_N, _H), jnp.float32)]
    if mode == "plain":
        scratch += [pltpu.VMEM_SHARED((_NS, _N), jnp.float32)]
    scratch += [pltpu.SemaphoreType.DMA, pltpu.SemaphoreType.DMA]
    part = jax.ShapeDtypeStruct((_N, _H), jnp.float32)
    out_type = [part, part]
    if mode == "plain":
        degp = jax.ShapeDtypeStruct((_N,), jnp.float32)
        out_type = [part, part, degp, degp]
    return pl.kernel(
        functools.partial(_spmm_body, mode),
        out_type=out_type,
        mesh=_MESH,
        scratch_types=scratch,
        compiler_params=pltpu.CompilerParams(
            needs_layout_passes=False, use_tc_tiling_on_sc=False),
    )


_spmm_attn = _make_spmm("attn")
_spmm_plain = _make_spmm("plain")
_spmm_plain0 = _make_spmm("plain0")


_BN = 1000
_GN = _N // _BN
_BM = 1024
_GM = -(-_N // _BM)
_BE = 16384
_GE = -(-_E // _BE)


def _row_spec(width, bn=_BN):
    return pl.BlockSpec((bn, width), lambda i: (i, 0))


def _vec_spec(bn=_BN):
    return pl.BlockSpec((bn,), lambda i: (i,))


def _const_spec(shape):
    nd = len(shape)
    return pl.BlockSpec(shape, lambda i: (0,) * nd)


def _dot(a, b):
    return jnp.dot(a, b, preferred_element_type=jnp.float32)


def _tca_body(x_ref, wne_ref, bne_ref, wl0_ref, bl0_ref, watt_ref, batt_ref,
              h_ref, hl_ref, ad_ref, as_ref):
    h = _dot(x_ref[...], wne_ref[...]) + bne_ref[...]
    hl = _dot(h, wl0_ref[...]) + bl0_ref[...]
    wa = watt_ref[...]
    h_ref[...] = h
    hl_ref[...] = hl
    ad_ref[...] = jnp.sum(hl * wa[0:_H, 0], axis=1) + batt_ref[...][0]
    as_ref[...] = jnp.sum(hl * wa[_H:2 * _H, 0], axis=1)


_tca = pl.pallas_call(
    _tca_body,
    grid=(_GM,),
    in_specs=[
        _row_spec(_H, _BM),
        _const_spec((_H, _H)), _const_spec((1, _H)),
        _const_spec((_H, _H)), _const_spec((1, _H)),
        _const_spec((2 * _H + _ED, 1)), _const_spec((1,)),
    ],
    out_specs=[_row_spec(_H, _BM), _row_spec(_H, _BM), _vec_spec(_BM),
               _vec_spec(_BM)],
    out_shape=[
        jax.ShapeDtypeStruct((_N, _H), jnp.float32),
        jax.ShapeDtypeStruct((_N, _H), jnp.float32),
        jax.ShapeDtypeStruct((_N,), jnp.float32),
        jax.ShapeDtypeStruct((_N,), jnp.float32),
    ],
)


def _tca2_body(ea_ref, watt_ref, ae_ref):
    ea = ea_ref[...]
    wa = watt_ref[...]
    ae_ref[...] = (ea[0] * wa[2 * _H, 0] + ea[1] * wa[2 * _H + 1, 0]
                   + ea[2] * wa[2 * _H + 2, 0])


_tca2 = pl.pallas_call(
    _tca2_body,
    grid=(_GE,),
    in_specs=[pl.BlockSpec((_ED, _BE), lambda i: (0, i)),
              _const_spec((2 * _H + _ED, 1))],
    out_specs=pl.BlockSpec((_BE,), lambda i: (i,)),
    out_shape=jax.ShapeDtypeStruct((_E,), jnp.float32),
)


def _stats_update(st_ref, v):
    @pl.when(pl.program_id(0) == 0)
    def _():
        st_ref[...] = jnp.zeros((8, _H), jnp.float32)

    upd = jnp.concatenate(
        [jnp.sum(v, axis=0)[None, :], jnp.sum(v * v, axis=0)[None, :],
         jnp.zeros((6, _H), jnp.float32)], axis=0)
    st_ref[...] += upd


def _bn_apply(st_ref, v, g, b):
    mu = st_ref[0:1, :] / _N
    var = st_ref[1:2, :] / _N - mu * mu
    return g * (v - mu) * lax.rsqrt(var + 1e-5) + b


def _tcb1_body(pa_ref, pb_ref, s_ref, st_ref):
    v = pa_ref[...] + pb_ref[...]
    s_ref[...] = v
    _stats_update(st_ref, v)


_tcb1 = pl.pallas_call(
    _tcb1_body,
    grid=(_GN,),
    in_specs=[_row_spec(_H), _row_spec(_H)],
    out_specs=[_row_spec(_H), _const_spec((8, _H))],
    out_shape=[jax.ShapeDtypeStruct((_N, _H), jnp.float32),
               jax.ShapeDtypeStruct((8, _H), jnp.float32)],
)


def _tcb2_body(s_ref, st_ref, h_ref, g_ref, b_ref, hb_ref):
    y = _bn_apply(st_ref, s_ref[...], g_ref[...], b_ref[...])
    hb_ref[...] = jnp.maximum(y, 0.0) + h_ref[...]


_tcb2 = pl.pallas_call(
    _tcb2_body,
    grid=(_GN,),
    in_specs=[_row_spec(_H), _const_spec((8, _H)), _row_spec(_H),
              _const_spec((1, _H)), _const_spec((1, _H))],
    out_specs=_row_spec(_H),
    out_shape=jax.ShapeDtypeStruct((_N, _H), jnp.float32),
)


def _tcc1_body(pa_ref, pb_ref, dega_ref, degb_ref, hb_ref, wsl_ref, bsl_ref,
               wsr_ref, t_ref, dinvc_ref, dinvf_ref, st_ref):
    ssum = pa_ref[...] + pb_ref[...]
    deg = dega_ref[...] + degb_ref[...]
    agg = ssum / jnp.maximum(deg, 1.0)[:, None]
    t = _dot(agg, wsl_ref[...]) + bsl_ref[...] + _dot(hb_ref[...], wsr_ref[...])
    t_ref[...] = t
    dinv = lax.rsqrt(deg + 1.0)
    dinvc_ref[...] = dinv[:, None]
    dinvf_ref[...] = dinv
    rows = pl.program_id(0) * _BM + lax.broadcasted_iota(jnp.int32, (_BM, 1), 0)
    _stats_update(st_ref, jnp.where(rows < _N, t, 0.0))


_tcc1 = pl.pallas_call(
    _tcc1_body,
    grid=(_GM,),
    in_specs=[_row_spec(_H, _BM), _row_spec(_H, _BM),
              _vec_spec(_BM), _vec_spec(_BM),
              _row_spec(_H, _BM), _const_spec((_H, _H)), _const_spec((1, _H)),
              _const_spec((_H, _H))],
    out_specs=[_row_spec(_H, _BM), _row_spec(1, _BM), _vec_spec(_BM),
               _const_spec((8, _H))],
    out_shape=[jax.ShapeDtypeStruct((_N, _H), jnp.float32),
               jax.ShapeDtypeStruct((_N, 1), jnp.float32),
               jax.ShapeDtypeStruct((_N,), jnp.float32),
               jax.ShapeDtypeStruct((8, _H), jnp.float32)],
)


def _tcc2_body(t_ref, st_ref, hb_ref, dinv_ref, g_ref, b_ref, wgcn_ref,
               hc_ref, hwp_ref):
    y = _bn_apply(st_ref, t_ref[...], g_ref[...], b_ref[...])
    hc = jnp.maximum(y, 0.0) + hb_ref[...]
    hc_ref[...] = hc
    hwp_ref[...] = _dot(hc, wgcn_ref[...]) * dinv_ref[...]


_tcc2 = pl.pallas_call(
    _tcc2_body,
    grid=(_GN,),
    in_specs=[_row_spec(_H), _const_spec((8, _H)), _row_spec(_H),
              _row_spec(1),
              _const_spec((1, _H)), _const_spec((1, _H)), _const_spec((_H, _H))],
    out_specs=[_row_spec(_H), _row_spec(_H)],
    out_shape=[jax.ShapeDtypeStruct((_N, _H), jnp.float32),
               jax.ShapeDtypeStruct((_N, _H), jnp.float32)],
)


def _tcd1_body(pa_ref, pb_ref, hwp_ref, dinv_ref, bgcn_ref, s_ref, st_ref):
    dinv = dinv_ref[...]
    v = (pa_ref[...] + pb_ref[...] + hwp_ref[...]) * dinv + bgcn_ref[...]
    s_ref[...] = v
    _stats_update(st_ref, v)


_tcd1 = pl.pallas_call(
    _tcd1_body,
    grid=(_GN,),
    in_specs=[_row_spec(_H), _row_spec(_H),
              _row_spec(_H), _row_spec(1), _const_spec((1, _H))],
    out_specs=[_row_spec(_H), _const_spec((8, _H))],
    out_shape=[jax.ShapeDtypeStruct((_N, _H), jnp.float32),
               jax.ShapeDtypeStruct((8, _H), jnp.float32)],
)


def _tcd2_body(s_ref, st_ref, hc_ref, g_ref, b_ref, wr1_ref, br1_ref,
               wr2_ref, br2_ref, wr3_ref, br3_ref, out_ref):
    y = _bn_apply(st_ref, s_ref[...], g_ref[...], b_ref[...])
    h = jnp.maximum(y, 0.0) + hc_ref[...]
    r = jnp.maximum(_dot(h, wr1_ref[...]) + br1_ref[...], 0.0)
    r = jnp.maximum(_dot(r, wr2_ref[...]) + br2_ref[...], 0.0)
    out_ref[...] = _dot(r, wr3_ref[...]) + br3_ref[...]


_tcd2 = pl.pallas_call(
    _tcd2_body,
    grid=(_GN,),
    in_specs=[_row_spec(_H), _const_spec((8, _H)), _row_spec(_H),
              _const_spec((1, _H)), _const_spec((1, _H)),
              _const_spec((_H, _H)), _const_spec((1, _H)),
              _const_spec((_H, _H // 2)), _const_spec((1, _H // 2)),
              _const_spec((_H // 2, 1)), _const_spec((1, 1))],
    out_specs=_row_spec(1),
    out_shape=jax.ShapeDtypeStruct((_N, 1), jnp.float32),
)


def kernel(x, edge_index, edge_attr, W_ne, b_ne, W_ee, b_ee, W_lin0, b_lin0,
           W_att, b_att, bn0_g, bn0_b, W_sl, b_sl, W_sr, bn1_g, bn1_b,
           W_gcn, b_gcn, bn2_g, bn2_b, W_r1, b_r1, W_r2, b_r2, W_r3, b_r3):
    del W_ee, b_ee
    ei = edge_index.reshape(2, _NW, _NCHUNK, _CH)
    row1 = lambda v: v.reshape(1, -1)
    zrows = jnp.zeros((_SLC, _H), jnp.float32)

    h, hl, a_d, a_s = _tca(x, W_ne, row1(b_ne), W_lin0, row1(b_lin0),
                           W_att, b_att)
    ae = _tca2(edge_attr.T, W_att)

    p0a, p0b = _spmm_attn(ei, ae, a_d, a_s, hl, zrows)
    s0, st0 = _tcb1(p0a, p0b)
    hb = _tcb2(s0, st0, h, row1(bn0_g), row1(bn0_b))

    p1a, p1b, dega, degb = _spmm_plain(ei, hb, zrows)
    t, dinv_c, dinv_f, st1 = _tcc1(p1a, p1b, dega, degb, hb,
                                   W_sl, row1(b_sl), W_sr)
    del dinv_f
    hc, hwp = _tcc2(t, st1, hb, dinv_c, row1(bn1_g), row1(bn1_b), W_gcn)

    p2a, p2b = _spmm_plain0(ei, hwp, zrows)
    s2, st2 = _tcd1(p2a, p2b, hwp, dinv_c, row1(b_gcn))
    out = _tcd2(s2, st2, hc, row1(bn2_g), row1(bn2_b), W_r1, row1(b_r1),
                W_r2, row1(b_r2), W_r3, row1(b_r3))
    return out
